# Initial kernel scaffold; baseline (speedup 1.0000x reference)
import functools

import jax
import jax.numpy as jnp
from jax import lax
from jax.experimental import pallas as pl
from jax.experimental.pallas import tpu as pltpu
from jax.experimental.pallas import tpu_sc as plsc

_NUM_ENTITY = 100000
_NUM_NEIGHBOR = 16
_BATCH = 4096
_NUM_CORES = 2
_NUM_SUBCORES = 16
_NUM_WORKERS = _NUM_CORES * _NUM_SUBCORES
_SEEDS_PER_W = _BATCH // _NUM_WORKERS


def _rf_body(seeds_hbm, adj_e_hbm, adj_r_hbm,
             o_e1, o_r1, o_e2, o_r2,
             seed_v, e1_v, r1_v, e2_v, r2_v,
             sem_e1, sem_r1, sem_e2, sem_r2):
    wid = lax.axis_index("s") * _NUM_CORES + lax.axis_index("c")
    base = wid * _SEEDS_PER_W

    pltpu.sync_copy(seeds_hbm.at[pl.ds(base, _SEEDS_PER_W)], seed_v)

    c_e1 = pltpu.async_copy(adj_e_hbm.at[seed_v], e1_v, sem_e1)
    c_r1 = pltpu.async_copy(adj_r_hbm.at[seed_v], r1_v, sem_r1)
    c_e1.wait()

    c_e2 = pltpu.async_copy(adj_e_hbm.at[e1_v], e2_v, sem_e2)
    c_r2 = pltpu.async_copy(adj_r_hbm.at[e1_v], r2_v, sem_r2)

    pltpu.sync_copy(e1_v, o_e1.at[pl.ds(base, _SEEDS_PER_W)])
    c_r1.wait()
    pltpu.sync_copy(r1_v, o_r1.at[pl.ds(base, _SEEDS_PER_W)])

    c_e2.wait()
    pltpu.sync_copy(e2_v, o_e2.at[pl.ds(base, _SEEDS_PER_W)])
    c_r2.wait()
    pltpu.sync_copy(r2_v, o_r2.at[pl.ds(base, _SEEDS_PER_W)])


_i32 = jnp.int32
_rf_call = functools.partial(
    pl.kernel,
    out_type=(
        jax.ShapeDtypeStruct((_BATCH, _NUM_NEIGHBOR), _i32),
        jax.ShapeDtypeStruct((_BATCH, _NUM_NEIGHBOR), _i32),
        jax.ShapeDtypeStruct((_BATCH, _NUM_NEIGHBOR, _NUM_NEIGHBOR), _i32),
        jax.ShapeDtypeStruct((_BATCH, _NUM_NEIGHBOR, _NUM_NEIGHBOR), _i32),
    ),
    mesh=plsc.VectorSubcoreMesh(core_axis_name="c", subcore_axis_name="s"),
    scratch_types=[
        pltpu.VMEM((_SEEDS_PER_W,), _i32),
        pltpu.VMEM((_SEEDS_PER_W, _NUM_NEIGHBOR), _i32),
        pltpu.VMEM((_SEEDS_PER_W, _NUM_NEIGHBOR), _i32),
        pltpu.VMEM((_SEEDS_PER_W, _NUM_NEIGHBOR, _NUM_NEIGHBOR), _i32),
        pltpu.VMEM((_SEEDS_PER_W, _NUM_NEIGHBOR, _NUM_NEIGHBOR), _i32),
        pltpu.SemaphoreType.DMA,
        pltpu.SemaphoreType.DMA,
        pltpu.SemaphoreType.DMA,
        pltpu.SemaphoreType.DMA,
    ],
)(_rf_body)


def kernel(inputs, adj_entity, adj_relation):
    seeds = inputs.reshape(_BATCH)
    e1, r1, e2, r2 = _rf_call(seeds, adj_entity, adj_relation)
    n2 = _NUM_NEIGHBOR * _NUM_NEIGHBOR
    return (
        (inputs, e1, e2.reshape(_BATCH, n2)),
        (r1, r2.reshape(_BATCH, n2)),
    )


# speedup vs baseline: 1.3578x; 1.3578x over previous
import functools

import jax
import jax.numpy as jnp
from jax import lax
from jax.experimental import pallas as pl
from jax.experimental.pallas import tpu as pltpu
from jax.experimental.pallas import tpu_sc as plsc

_NUM_ENTITY = 100000
_NUM_NEIGHBOR = 16
_BATCH = 4096
_NUM_CORES = 2
_NUM_SUBCORES = 16
_NUM_WORKERS = _NUM_CORES * _NUM_SUBCORES
_SEEDS_PER_W = _BATCH // _NUM_WORKERS
_HOP2_PER_W = _SEEDS_PER_W * _NUM_NEIGHBOR
_CHUNK = 128
_N_CHUNKS = _HOP2_PER_W // _CHUNK


def _rf_body(seeds_hbm, adj_e_hbm, adj_r_hbm,
             o_e1, o_r1, o_e2, o_r2,
             seed_v, e1_v, r1_v, idx2_v, e2_v, r2_v,
             sem_e1, sem_r1, sem_e2, sem_r2):
    wid = lax.axis_index("s") * _NUM_CORES + lax.axis_index("c")
    base = wid * _SEEDS_PER_W

    pltpu.sync_copy(seeds_hbm.at[pl.ds(base, _SEEDS_PER_W)], seed_v)

    c_e1 = pltpu.async_copy(adj_e_hbm.at[seed_v], e1_v, sem_e1)
    c_r1 = pltpu.async_copy(adj_r_hbm.at[seed_v], r1_v, sem_r1)
    c_e1.wait()

    def _flat(i, _):
        idx2_v[pl.ds(i * _NUM_NEIGHBOR, _NUM_NEIGHBOR)] = e1_v[i, :]
        return 0

    lax.fori_loop(0, _SEEDS_PER_W, _flat, 0)

    for j in range(_N_CHUNKS):
        idx = idx2_v.at[pl.ds(j * _CHUNK, _CHUNK)]
        pltpu.async_copy(adj_e_hbm.at[idx], e2_v.at[pl.ds(j * _CHUNK, _CHUNK)], sem_e2)
        pltpu.async_copy(adj_r_hbm.at[idx], r2_v.at[pl.ds(j * _CHUNK, _CHUNK)], sem_r2)

    pltpu.sync_copy(e1_v, o_e1.at[pl.ds(base, _SEEDS_PER_W)])
    c_r1.wait()
    pltpu.sync_copy(r1_v, o_r1.at[pl.ds(base, _SEEDS_PER_W)])

    for j in range(_N_CHUNKS):
        idx = idx2_v.at[pl.ds(j * _CHUNK, _CHUNK)]
        pltpu.make_async_copy(adj_e_hbm.at[idx], e2_v.at[pl.ds(j * _CHUNK, _CHUNK)], sem_e2).wait()
    pltpu.sync_copy(e2_v, o_e2.at[pl.ds(wid * _HOP2_PER_W, _HOP2_PER_W)])
    for j in range(_N_CHUNKS):
        idx = idx2_v.at[pl.ds(j * _CHUNK, _CHUNK)]
        pltpu.make_async_copy(adj_r_hbm.at[idx], r2_v.at[pl.ds(j * _CHUNK, _CHUNK)], sem_r2).wait()
    pltpu.sync_copy(r2_v, o_r2.at[pl.ds(wid * _HOP2_PER_W, _HOP2_PER_W)])


_i32 = jnp.int32
_N_HOP2 = _BATCH * _NUM_NEIGHBOR
_rf_call = functools.partial(
    pl.kernel,
    out_type=(
        jax.ShapeDtypeStruct((_BATCH, _NUM_NEIGHBOR), _i32),
        jax.ShapeDtypeStruct((_BATCH, _NUM_NEIGHBOR), _i32),
        jax.ShapeDtypeStruct((_N_HOP2, _NUM_NEIGHBOR), _i32),
        jax.ShapeDtypeStruct((_N_HOP2, _NUM_NEIGHBOR), _i32),
    ),
    mesh=plsc.VectorSubcoreMesh(core_axis_name="c", subcore_axis_name="s"),
    compiler_params=pltpu.CompilerParams(use_tc_tiling_on_sc=False),
    scratch_types=[
        pltpu.VMEM((_SEEDS_PER_W,), _i32),
        pltpu.VMEM((_SEEDS_PER_W, _NUM_NEIGHBOR), _i32),
        pltpu.VMEM((_SEEDS_PER_W, _NUM_NEIGHBOR), _i32),
        pltpu.VMEM((_HOP2_PER_W,), _i32),
        pltpu.VMEM((_HOP2_PER_W, _NUM_NEIGHBOR), _i32),
        pltpu.VMEM((_HOP2_PER_W, _NUM_NEIGHBOR), _i32),
        pltpu.SemaphoreType.DMA,
        pltpu.SemaphoreType.DMA,
        pltpu.SemaphoreType.DMA,
        pltpu.SemaphoreType.DMA,
    ],
)(_rf_body)


def kernel(inputs, adj_entity, adj_relation):
    seeds = inputs.reshape(_BATCH)
    e1, r1, e2, r2 = _rf_call(seeds, adj_entity, adj_relation)
    n2 = _NUM_NEIGHBOR * _NUM_NEIGHBOR
    return (
        (inputs, e1, e2.reshape(_BATCH, n2)),
        (r1, r2.reshape(_BATCH, n2)),
    )


# speedup vs baseline: 1.3580x; 1.0001x over previous
import functools

import jax
import jax.numpy as jnp
from jax import lax
from jax.experimental import pallas as pl
from jax.experimental.pallas import tpu as pltpu
from jax.experimental.pallas import tpu_sc as plsc

_NUM_ENTITY = 100000
_NUM_NEIGHBOR = 16
_BATCH = 4096
_NUM_CORES = 2
_NUM_SUBCORES = 16
_NUM_WORKERS = _NUM_CORES * _NUM_SUBCORES
_SEEDS_PER_W = _BATCH // _NUM_WORKERS
_HOP2_PER_W = _SEEDS_PER_W * _NUM_NEIGHBOR
_CHUNK = 2048
_N_CHUNKS = _HOP2_PER_W // _CHUNK


def _rf_body(seeds_hbm, adj_e_hbm, adj_r_hbm,
             o_e1, o_r1, o_e2, o_r2,
             seed_v, e1_v, r1_v, idx2_v, e2_v, r2_v,
             sem_e1, sem_r1, sem_e2, sem_r2):
    wid = lax.axis_index("s") * _NUM_CORES + lax.axis_index("c")
    base = wid * _SEEDS_PER_W

    pltpu.sync_copy(seeds_hbm.at[pl.ds(base, _SEEDS_PER_W)], seed_v)

    c_e1 = pltpu.async_copy(adj_e_hbm.at[seed_v], e1_v, sem_e1)
    c_r1 = pltpu.async_copy(adj_r_hbm.at[seed_v], r1_v, sem_r1)
    c_e1.wait()

    def _flat(i, _):
        idx2_v[pl.ds(i * _NUM_NEIGHBOR, _NUM_NEIGHBOR)] = e1_v[i, :]
        return 0

    lax.fori_loop(0, _SEEDS_PER_W, _flat, 0)

    for j in range(_N_CHUNKS):
        idx = idx2_v.at[pl.ds(j * _CHUNK, _CHUNK)]
        pltpu.async_copy(adj_e_hbm.at[idx], e2_v.at[pl.ds(j * _CHUNK, _CHUNK)], sem_e2)
        pltpu.async_copy(adj_r_hbm.at[idx], r2_v.at[pl.ds(j * _CHUNK, _CHUNK)], sem_r2)

    pltpu.sync_copy(e1_v, o_e1.at[pl.ds(base, _SEEDS_PER_W)])
    c_r1.wait()
    pltpu.sync_copy(r1_v, o_r1.at[pl.ds(base, _SEEDS_PER_W)])

    for j in range(_N_CHUNKS):
        idx = idx2_v.at[pl.ds(j * _CHUNK, _CHUNK)]
        pltpu.make_async_copy(adj_e_hbm.at[idx], e2_v.at[pl.ds(j * _CHUNK, _CHUNK)], sem_e2).wait()
    pltpu.sync_copy(e2_v, o_e2.at[pl.ds(wid * _HOP2_PER_W, _HOP2_PER_W)])
    for j in range(_N_CHUNKS):
        idx = idx2_v.at[pl.ds(j * _CHUNK, _CHUNK)]
        pltpu.make_async_copy(adj_r_hbm.at[idx], r2_v.at[pl.ds(j * _CHUNK, _CHUNK)], sem_r2).wait()
    pltpu.sync_copy(r2_v, o_r2.at[pl.ds(wid * _HOP2_PER_W, _HOP2_PER_W)])


_i32 = jnp.int32
_N_HOP2 = _BATCH * _NUM_NEIGHBOR
_rf_call = functools.partial(
    pl.kernel,
    out_type=(
        jax.ShapeDtypeStruct((_BATCH, _NUM_NEIGHBOR), _i32),
        jax.ShapeDtypeStruct((_BATCH, _NUM_NEIGHBOR), _i32),
        jax.ShapeDtypeStruct((_N_HOP2, _NUM_NEIGHBOR), _i32),
        jax.ShapeDtypeStruct((_N_HOP2, _NUM_NEIGHBOR), _i32),
    ),
    mesh=plsc.VectorSubcoreMesh(core_axis_name="c", subcore_axis_name="s"),
    compiler_params=pltpu.CompilerParams(use_tc_tiling_on_sc=False),
    scratch_types=[
        pltpu.VMEM((_SEEDS_PER_W,), _i32),
        pltpu.VMEM((_SEEDS_PER_W, _NUM_NEIGHBOR), _i32),
        pltpu.VMEM((_SEEDS_PER_W, _NUM_NEIGHBOR), _i32),
        pltpu.VMEM((_HOP2_PER_W,), _i32),
        pltpu.VMEM((_HOP2_PER_W, _NUM_NEIGHBOR), _i32),
        pltpu.VMEM((_HOP2_PER_W, _NUM_NEIGHBOR), _i32),
        pltpu.SemaphoreType.DMA,
        pltpu.SemaphoreType.DMA,
        pltpu.SemaphoreType.DMA,
        pltpu.SemaphoreType.DMA,
    ],
)(_rf_body)


def kernel(inputs, adj_entity, adj_relation):
    seeds = inputs.reshape(_BATCH)
    e1, r1, e2, r2 = _rf_call(seeds, adj_entity, adj_relation)
    n2 = _NUM_NEIGHBOR * _NUM_NEIGHBOR
    return (
        (inputs, e1, e2.reshape(_BATCH, n2)),
        (r1, r2.reshape(_BATCH, n2)),
    )


# speedup vs baseline: 1.3588x; 1.0006x over previous
import functools

import jax
import jax.numpy as jnp
from jax import lax
from jax.experimental import pallas as pl
from jax.experimental.pallas import tpu as pltpu
from jax.experimental.pallas import tpu_sc as plsc

_NUM_ENTITY = 100000
_NUM_NEIGHBOR = 16
_BATCH = 4096
_NUM_CORES = 2
_NUM_SUBCORES = 16
_NUM_WORKERS = _NUM_CORES * _NUM_SUBCORES
_SEEDS_PER_W = _BATCH // _NUM_WORKERS
_HOP2_PER_W = _SEEDS_PER_W * _NUM_NEIGHBOR


def _rf_body(seeds_hbm, adj_e_hbm, adj_r_hbm,
             o_e1, o_r1, o_e2, o_r2,
             seed_v, e1_v, r1_v, idx2_v, e2_v, r2_v,
             sem_e1, sem_r1, sem_e2, sem_r2):
    wid = lax.axis_index("s") * _NUM_CORES + lax.axis_index("c")
    base = wid * _SEEDS_PER_W

    pltpu.sync_copy(seeds_hbm.at[pl.ds(base, _SEEDS_PER_W)], seed_v)

    c_e1 = pltpu.async_copy(adj_e_hbm.at[seed_v], e1_v, sem_e1)
    c_r1 = pltpu.async_copy(adj_r_hbm.at[seed_v], r1_v, sem_r1)
    c_e1.wait()

    def _flat(i, _):
        idx2_v[pl.ds(i * _NUM_NEIGHBOR, _NUM_NEIGHBOR)] = e1_v[i, :]
        return 0

    lax.fori_loop(0, _SEEDS_PER_W, _flat, 0)

    c_e2 = pltpu.async_copy(adj_e_hbm.at[idx2_v], e2_v, sem_e2)
    c_r2 = pltpu.async_copy(adj_r_hbm.at[idx2_v], r2_v, sem_r2)

    pltpu.sync_copy(e1_v, o_e1.at[pl.ds(base, _SEEDS_PER_W)])
    c_r1.wait()
    pltpu.sync_copy(r1_v, o_r1.at[pl.ds(base, _SEEDS_PER_W)])

    c_e2.wait()
    pltpu.sync_copy(e2_v, o_e2.at[pl.ds(wid * _HOP2_PER_W, _HOP2_PER_W)])
    c_r2.wait()
    pltpu.sync_copy(r2_v, o_r2.at[pl.ds(wid * _HOP2_PER_W, _HOP2_PER_W)])


_i32 = jnp.int32
_N_HOP2 = _BATCH * _NUM_NEIGHBOR
_rf_call = functools.partial(
    pl.kernel,
    out_type=(
        jax.ShapeDtypeStruct((_BATCH, _NUM_NEIGHBOR), _i32),
        jax.ShapeDtypeStruct((_BATCH, _NUM_NEIGHBOR), _i32),
        jax.ShapeDtypeStruct((_N_HOP2, _NUM_NEIGHBOR), _i32),
        jax.ShapeDtypeStruct((_N_HOP2, _NUM_NEIGHBOR), _i32),
    ),
    mesh=plsc.VectorSubcoreMesh(core_axis_name="c", subcore_axis_name="s"),
    compiler_params=pltpu.CompilerParams(
        use_tc_tiling_on_sc=False,
        skip_device_barrier=True,
        disable_bounds_checks=True,
        disable_semaphore_checks=True,
    ),
    scratch_types=[
        pltpu.VMEM((_SEEDS_PER_W,), _i32),
        pltpu.VMEM((_SEEDS_PER_W, _NUM_NEIGHBOR), _i32),
        pltpu.VMEM((_SEEDS_PER_W, _NUM_NEIGHBOR), _i32),
        pltpu.VMEM((_HOP2_PER_W,), _i32),
        pltpu.VMEM((_HOP2_PER_W, _NUM_NEIGHBOR), _i32),
        pltpu.VMEM((_HOP2_PER_W, _NUM_NEIGHBOR), _i32),
        pltpu.SemaphoreType.DMA,
        pltpu.SemaphoreType.DMA,
        pltpu.SemaphoreType.DMA,
        pltpu.SemaphoreType.DMA,
    ],
)(_rf_body)


def kernel(inputs, adj_entity, adj_relation):
    seeds = inputs.reshape(_BATCH)
    e1, r1, e2, r2 = _rf_call(seeds, adj_entity, adj_relation)
    n2 = _NUM_NEIGHBOR * _NUM_NEIGHBOR
    return (
        (inputs, e1, e2.reshape(_BATCH, n2)),
        (r1, r2.reshape(_BATCH, n2)),
    )
